# baseline (speedup 1.0000x reference)
import jax
import jax.numpy as jnp
from jax.experimental import pallas as pl

N_GRAPHS = 64
P = 128
K = 16
N = N_GRAPHS * P
EPS = 1e-5
GB = 2


def _bn_st(sums, gamma, beta, count):
    mean = sums[0:1] / count
    var = sums[1:2] / count - mean * mean
    s = gamma * jax.lax.rsqrt(var + EPS)
    return s, beta - mean * s


def _accum(ref, value, step):
    @pl.when(step == 0)
    def _():
        ref[...] = value

    @pl.when(step > 0)
    def _():
        ref[...] = ref[...] + value


def _in_stats_body(x_ref, o_ref):
    xb = x_ref[...]
    o_ref[...] = jnp.stack([jnp.sum(xb, 0), jnp.sum(xb * xb, 0)])


def _input_stats(x):
    return pl.pallas_call(
        _in_stats_body,
        out_shape=jax.ShapeDtypeStruct((2, x.shape[1]), jnp.float32),
    )(x)


def _knn_edges_and_stats(fts, pts, wd, wb, e_ref, es_ref, step):
    A = jnp.dot(fts, wd)
    B = jnp.dot(fts, wb).astype(jnp.bfloat16)
    rows = jax.lax.broadcasted_iota(jnp.int32, (P, P), 0)
    cols = jax.lax.broadcasted_iota(jnp.int32, (P, P), 1)
    keys = []
    for g in range(GB):
        p_g = pts[g * P:(g + 1) * P]
        n2 = jnp.sum(p_g * p_g, axis=1, keepdims=True)
        pp = jnp.dot(p_g, p_g.T)
        d = jnp.maximum(n2 + n2.T - 2.0 * pp, 0.0)
        d = jnp.where(rows == cols, d + 1e12, d)
        kb = (jax.lax.bitcast_convert_type(d, jnp.int32) & ~127) | cols
        keys.append(jax.lax.bitcast_convert_type(kb, jnp.float32))
    esum = jnp.zeros((P, A.shape[1]), jnp.float32)
    esq = jnp.zeros((P, A.shape[1]), jnp.float32)
    for t in range(K):
        for g in range(GB):
            key = keys[g]
            m = jnp.min(key, axis=1, keepdims=True)
            oh = key == m
            e_t = A[g * P:(g + 1) * P] + jnp.dot(
                oh.astype(jnp.bfloat16), B[g * P:(g + 1) * P],
                preferred_element_type=jnp.float32)
            e_ref[t, g * P:(g + 1) * P, :] = e_t.astype(jnp.bfloat16)
            esum = esum + e_t
            esq = esq + e_t * e_t
            keys[g] = jnp.where(oh, jnp.inf, key)
    stats = jnp.stack([jnp.sum(esum, 0), jnp.sum(esq, 0)])
    _accum(es_ref, stats, step)


def _small_specs(chs):
    return [pl.BlockSpec((2, c) if two else (1, c), lambda g: (0, 0))
            for c, two in chs]


def _l1_body(sin_ref, g0_ref, b0_ref, pos_ref, x_ref, wd_ref, wb_ref, ws_ref,
             e_ref, skip_ref, es_ref, ns_ref):
    g = pl.program_id(0)
    s0, t0 = _bn_st(sin_ref[...], g0_ref[...], b0_ref[...], float(N))
    fts = x_ref[...] * s0 + t0
    skip = jnp.dot(fts, ws_ref[...])
    skip_ref[...] = skip
    _accum(ns_ref, jnp.stack([jnp.sum(skip, 0), jnp.sum(skip * skip, 0)]), g)
    _knn_edges_and_stats(fts, pos_ref[...], wd_ref[...], wb_ref[...], e_ref, es_ref, g)


def _layer1_entry(x, pos, sin, g0, b0, wd, wb, ws):
    ch1, ch3 = wd.shape[1], ws.shape[1]
    f = x.shape[1]
    pdim = pos.shape[1]
    return pl.pallas_call(
        _l1_body,
        grid=(N_GRAPHS // GB,),
        in_specs=_small_specs([(f, True), (f, False), (f, False)]) + [
            pl.BlockSpec((GB * P, pdim), lambda g: (g, 0)),
            pl.BlockSpec((GB * P, f), lambda g: (g, 0)),
            pl.BlockSpec((f, ch1), lambda g: (0, 0)),
            pl.BlockSpec((f, ch1), lambda g: (0, 0)),
            pl.BlockSpec((f, ch3), lambda g: (0, 0)),
        ],
        out_specs=[
            pl.BlockSpec((K, GB * P, ch1), lambda g: (0, g, 0)),
            pl.BlockSpec((GB * P, ch3), lambda g: (g, 0)),
            pl.BlockSpec((2, ch1), lambda g: (0, 0)),
            pl.BlockSpec((2, ch3), lambda g: (0, 0)),
        ],
        out_shape=[
            jax.ShapeDtypeStruct((K, N, ch1), jnp.bfloat16),
            jax.ShapeDtypeStruct((N, ch3), jnp.float32),
            jax.ShapeDtypeStruct((2, ch1), jnp.float32),
            jax.ShapeDtypeStruct((2, ch3), jnp.float32),
        ],
    )(sin, g0, b0, pos, x, wd, wb, ws)


def _mid_body(sums_ref, g_ref, b_ref, e_ref, w_ref, o_ref, st_ref):
    i = pl.program_id(0)
    s, t = _bn_st(sums_ref[...], g_ref[...], b_ref[...], float(N * K))
    h = jnp.maximum(e_ref[...].astype(jnp.float32) * s + t, 0.0)
    o = jnp.dot(h, w_ref[...])
    o_ref[...] = o.astype(jnp.bfloat16)
    _accum(st_ref, jnp.stack([jnp.sum(o, 0), jnp.sum(o * o, 0)]), i)


def _mid_layer(e_flat, sums, gamma, beta, w, block_rows=16384):
    rows, chp = e_flat.shape
    ch = w.shape[1]
    return pl.pallas_call(
        _mid_body,
        grid=(rows // block_rows,),
        in_specs=_small_specs([(chp, True), (chp, False), (chp, False)]) + [
            pl.BlockSpec((block_rows, chp), lambda i: (i, 0)),
            pl.BlockSpec((chp, ch), lambda i: (0, 0)),
        ],
        out_specs=[
            pl.BlockSpec((block_rows, ch), lambda i: (i, 0)),
            pl.BlockSpec((2, ch), lambda i: (0, 0)),
        ],
        out_shape=[
            jax.ShapeDtypeStruct((rows, ch), jnp.bfloat16),
            jax.ShapeDtypeStruct((2, ch), jnp.float32),
        ],
    )(sums, gamma, beta, e_flat, w)


def _mid_max_body(sums_ref, g_ref, b_ref, e_ref, w_ref, m_ref, st_ref):
    i = pl.program_id(0)
    s, t = _bn_st(sums_ref[...], g_ref[...], b_ref[...], float(N * K))
    h = jnp.maximum(e_ref[...].astype(jnp.float32) * s + t, 0.0)
    o = jnp.dot(h, w_ref[...])
    om = jnp.max(o.reshape(-1, N, o.shape[1]), axis=0).astype(jnp.bfloat16)

    @pl.when(i == 0)
    def _():
        m_ref[...] = om

    @pl.when(i > 0)
    def _():
        m_ref[...] = jnp.maximum(m_ref[...], om)

    _accum(st_ref, jnp.stack([jnp.sum(o, 0), jnp.sum(o * o, 0)]), i)


def _mid_max_layer(e_flat, sums, gamma, beta, w, block_rows=16384):
    rows, chp = e_flat.shape
    ch = w.shape[1]
    return pl.pallas_call(
        _mid_max_body,
        grid=(rows // block_rows,),
        in_specs=_small_specs([(chp, True), (chp, False), (chp, False)]) + [
            pl.BlockSpec((block_rows, chp), lambda i: (i, 0)),
            pl.BlockSpec((chp, ch), lambda i: (0, 0)),
        ],
        out_specs=[
            pl.BlockSpec((N, ch), lambda i: (0, 0)),
            pl.BlockSpec((2, ch), lambda i: (0, 0)),
        ],
        out_shape=[
            jax.ShapeDtypeStruct((N, ch), jnp.bfloat16),
            jax.ShapeDtypeStruct((2, ch), jnp.float32),
        ],
    )(sums, gamma, beta, e_flat, w)


def _da_body(es3_ref, g3_ref, b3_ref, ns_ref_in, gs_ref, bs_ref,
             m3_ref, skip_ref, wd_ref, wb_ref, ws_ref,
             e_ref, skip_out_ref, es_ref, ns_ref):
    g = pl.program_id(0)
    s3, t3 = _bn_st(es3_ref[...], g3_ref[...], b3_ref[...], float(N * K))
    ss, ts = _bn_st(ns_ref_in[...], gs_ref[...], bs_ref[...], float(N))
    aggr = jnp.maximum(m3_ref[...].astype(jnp.float32) * s3 + t3, 0.0)
    fts = jnp.maximum(aggr + skip_ref[...] * ss + ts, 0.0)
    skip = jnp.dot(fts, ws_ref[...])
    skip_out_ref[...] = skip
    _accum(ns_ref, jnp.stack([jnp.sum(skip, 0), jnp.sum(skip * skip, 0)]), g)
    _knn_edges_and_stats(fts, fts, wd_ref[...], wb_ref[...], e_ref, es_ref, g)


def _da_layer(es3, g3, b3, nss, gs, bs, m3, skip_prev, wd, wb, ws):
    chp = m3.shape[1]
    ch1, ch3 = wd.shape[1], ws.shape[1]
    return pl.pallas_call(
        _da_body,
        grid=(N_GRAPHS // GB,),
        in_specs=_small_specs([(chp, True), (chp, False), (chp, False),
                               (chp, True), (chp, False), (chp, False)]) + [
            pl.BlockSpec((GB * P, chp), lambda g: (g, 0)),
            pl.BlockSpec((GB * P, chp), lambda g: (g, 0)),
            pl.BlockSpec((chp, ch1), lambda g: (0, 0)),
            pl.BlockSpec((chp, ch1), lambda g: (0, 0)),
            pl.BlockSpec((chp, ch3), lambda g: (0, 0)),
        ],
        out_specs=[
            pl.BlockSpec((K, GB * P, ch1), lambda g: (0, g, 0)),
            pl.BlockSpec((GB * P, ch3), lambda g: (g, 0)),
            pl.BlockSpec((2, ch1), lambda g: (0, 0)),
            pl.BlockSpec((2, ch3), lambda g: (0, 0)),
        ],
        out_shape=[
            jax.ShapeDtypeStruct((K, N, ch1), jnp.bfloat16),
            jax.ShapeDtypeStruct((N, ch3), jnp.float32),
            jax.ShapeDtypeStruct((2, ch1), jnp.float32),
            jax.ShapeDtypeStruct((2, ch3), jnp.float32),
        ],
    )(es3, g3, b3, nss, gs, bs, m3, skip_prev, wd, wb, ws)


def _pool_body(es3_ref, g3_ref, b3_ref, ns_ref_in, gs_ref, bs_ref,
               m3_ref, skip_ref, o_ref):
    s3, t3 = _bn_st(es3_ref[...], g3_ref[...], b3_ref[...], float(N * K))
    ss, ts = _bn_st(ns_ref_in[...], gs_ref[...], bs_ref[...], float(N))
    aggr = jnp.maximum(m3_ref[...].astype(jnp.float32) * s3 + t3, 0.0)
    fts = jnp.maximum(aggr + skip_ref[...] * ss + ts, 0.0)
    o_ref[...] = jnp.mean(fts.reshape(GB, P, -1), axis=1, keepdims=True)


def _pool_layer(es3, g3, b3, nss, gs, bs, m3, skip_prev):
    chp = m3.shape[1]
    return pl.pallas_call(
        _pool_body,
        grid=(N_GRAPHS // GB,),
        in_specs=_small_specs([(chp, True), (chp, False), (chp, False),
                               (chp, True), (chp, False), (chp, False)]) + [
            pl.BlockSpec((GB * P, chp), lambda g: (g, 0)),
            pl.BlockSpec((GB * P, chp), lambda g: (g, 0)),
        ],
        out_specs=[pl.BlockSpec((GB, 1, chp), lambda g: (g, 0, 0))],
        out_shape=[jax.ShapeDtypeStruct((N_GRAPHS, 1, chp), jnp.float32)],
    )(es3, g3, b3, nss, gs, bs, m3, skip_prev)[0]


def _head_body(h_ref, wfc_ref, bfc_ref, wout_ref, bout_ref, o_ref):
    h = jnp.maximum(jnp.dot(h_ref[...], wfc_ref[...]) + bfc_ref[...], 0.0)
    logits = jnp.dot(h, wout_ref[...]) + bout_ref[...]
    m = jnp.max(logits, axis=1, keepdims=True)
    e = jnp.exp(logits - m)
    o_ref[...] = e / jnp.sum(e, axis=1, keepdims=True)


def _head(pooled, wfc, bfc, wout_pad, bout_pad):
    return pl.pallas_call(
        _head_body,
        out_shape=jax.ShapeDtypeStruct((N_GRAPHS, 128), jnp.float32),
    )(pooled, wfc, bfc, wout_pad, bout_pad)


def kernel(x, pos, batch, params):
    f = x.shape[1]
    convs = params['convs']

    prep = []
    prev = f
    for p in convs:
        w1 = p['W1']
        w1a, w1b = w1[:, :prev], w1[:, prev:]
        prep.append({
            'wd': (w1a - w1b).T, 'wb': w1b.T,
            'w2': p['W2'].T, 'w3': p['W3'].T, 'ws': p['Ws'].T,
            'g1': p['g1'].reshape(1, -1), 'b1': p['b1'].reshape(1, -1),
            'g2': p['g2'].reshape(1, -1), 'b2': p['b2'].reshape(1, -1),
            'g3': p['g3'].reshape(1, -1), 'b3': p['b3'].reshape(1, -1),
            'gs': p['gs'].reshape(1, -1), 'bs': p['bs'].reshape(1, -1),
        })
        prev = p['W3'].shape[0]

    sin = _input_stats(x)
    g0 = params['input_bn']['g'].reshape(1, -1)
    b0 = params['input_bn']['b'].reshape(1, -1)
    e1, skip, es, ns = _layer1_entry(x, pos, sin, g0, b0, prep[0]['wd'],
                                     prep[0]['wb'], prep[0]['ws'])
    for li in range(3):
        pr = prep[li]
        e2, es2 = _mid_layer(e1.reshape(N * K, -1), es, pr['g1'], pr['b1'],
                             pr['w2'])
        m3, es3 = _mid_max_layer(e2, es2, pr['g2'], pr['b2'], pr['w3'])
        if li < 2:
            e1, skip, es, ns = _da_layer(es3, pr['g3'], pr['b3'], ns,
                                         pr['gs'], pr['bs'], m3, skip,
                                         prep[li + 1]['wd'],
                                         prep[li + 1]['wb'],
                                         prep[li + 1]['ws'])
        else:
            pooled = _pool_layer(es3, pr['g3'], pr['b3'], ns,
                                 pr['gs'], pr['bs'], m3, skip)

    pooled = pooled.reshape(N_GRAPHS, -1)
    fp = params['fc'][0]
    wout = params['out']['W']
    wout_pad = jnp.zeros((wout.shape[1], 128), jnp.float32).at[:, :wout.shape[0]].set(wout.T)
    bout_pad = jnp.full((1, 128), -1e30, jnp.float32).at[0, :wout.shape[0]].set(params['out']['b'])
    probs = _head(pooled, fp['W'].T, fp['b'].reshape(1, -1), wout_pad, bout_pad)
    return probs[:, :wout.shape[0]]


# speedup vs baseline: 1.1836x; 1.1836x over previous
import jax
import jax.numpy as jnp
from jax.experimental import pallas as pl

N_GRAPHS = 64
P = 128
K = 16
N = N_GRAPHS * P
EPS = 1e-5
GB = 4


def _bn_st(sums, gamma, beta, count):
    mean = sums[0:1] / count
    var = sums[1:2] / count - mean * mean
    s = gamma * jax.lax.rsqrt(var + EPS)
    return s, beta - mean * s


def _accum(ref, value, step):
    @pl.when(step == 0)
    def _():
        ref[...] = value

    @pl.when(step > 0)
    def _():
        ref[...] = ref[...] + value


def _in_stats_body(x_ref, o_ref):
    xb = x_ref[...]
    o_ref[...] = jnp.stack([jnp.sum(xb, 0), jnp.sum(xb * xb, 0)])


def _input_stats(x):
    return pl.pallas_call(
        _in_stats_body,
        out_shape=jax.ShapeDtypeStruct((2, x.shape[1]), jnp.float32),
    )(x)


def _knn_edges_and_stats(fts, pts, wd, wb, e_ref, es_ref, step):
    A = jnp.dot(fts, wd)
    B = jnp.dot(fts, wb).astype(jnp.bfloat16)
    rows = jax.lax.broadcasted_iota(jnp.int32, (P, P), 0)
    cols = jax.lax.broadcasted_iota(jnp.int32, (P, P), 1)
    keys = []
    for g in range(GB):
        p_g = pts[g * P:(g + 1) * P]
        n2 = jnp.sum(p_g * p_g, axis=1, keepdims=True)
        pp = jnp.dot(p_g, p_g.T)
        d = jnp.maximum(n2 + n2.T - 2.0 * pp, 0.0)
        d = jnp.where(rows == cols, d + 1e12, d)
        kb = (jax.lax.bitcast_convert_type(d, jnp.int32) & ~127) | cols
        keys.append(jax.lax.bitcast_convert_type(kb, jnp.float32))
    esum = jnp.zeros((P, A.shape[1]), jnp.float32)
    esq = jnp.zeros((P, A.shape[1]), jnp.float32)
    for t in range(K):
        for g in range(GB):
            key = keys[g]
            m = jnp.min(key, axis=1, keepdims=True)
            oh = key == m
            e_t = A[g * P:(g + 1) * P] + jnp.dot(
                oh.astype(jnp.bfloat16), B[g * P:(g + 1) * P],
                preferred_element_type=jnp.float32)
            e_ref[t, g * P:(g + 1) * P, :] = e_t.astype(jnp.bfloat16)
            esum = esum + e_t
            esq = esq + e_t * e_t
            keys[g] = jnp.where(oh, jnp.inf, key)
    stats = jnp.stack([jnp.sum(esum, 0), jnp.sum(esq, 0)])
    _accum(es_ref, stats, step)


def _small_specs(chs):
    return [pl.BlockSpec((2, c) if two else (1, c), lambda g: (0, 0))
            for c, two in chs]


def _l1_body(sin_ref, g0_ref, b0_ref, pos_ref, x_ref, wd_ref, wb_ref, ws_ref,
             e_ref, skip_ref, es_ref, ns_ref):
    g = pl.program_id(0)
    s0, t0 = _bn_st(sin_ref[...], g0_ref[...], b0_ref[...], float(N))
    fts = x_ref[...] * s0 + t0
    skip = jnp.dot(fts, ws_ref[...])
    skip_ref[...] = skip
    _accum(ns_ref, jnp.stack([jnp.sum(skip, 0), jnp.sum(skip * skip, 0)]), g)
    _knn_edges_and_stats(fts, pos_ref[...], wd_ref[...], wb_ref[...], e_ref, es_ref, g)


def _layer1_entry(x, pos, sin, g0, b0, wd, wb, ws):
    ch1, ch3 = wd.shape[1], ws.shape[1]
    f = x.shape[1]
    pdim = pos.shape[1]
    return pl.pallas_call(
        _l1_body,
        grid=(N_GRAPHS // GB,),
        in_specs=_small_specs([(f, True), (f, False), (f, False)]) + [
            pl.BlockSpec((GB * P, pdim), lambda g: (g, 0)),
            pl.BlockSpec((GB * P, f), lambda g: (g, 0)),
            pl.BlockSpec((f, ch1), lambda g: (0, 0)),
            pl.BlockSpec((f, ch1), lambda g: (0, 0)),
            pl.BlockSpec((f, ch3), lambda g: (0, 0)),
        ],
        out_specs=[
            pl.BlockSpec((K, GB * P, ch1), lambda g: (0, g, 0)),
            pl.BlockSpec((GB * P, ch3), lambda g: (g, 0)),
            pl.BlockSpec((2, ch1), lambda g: (0, 0)),
            pl.BlockSpec((2, ch3), lambda g: (0, 0)),
        ],
        out_shape=[
            jax.ShapeDtypeStruct((K, N, ch1), jnp.bfloat16),
            jax.ShapeDtypeStruct((N, ch3), jnp.float32),
            jax.ShapeDtypeStruct((2, ch1), jnp.float32),
            jax.ShapeDtypeStruct((2, ch3), jnp.float32),
        ],
    )(sin, g0, b0, pos, x, wd, wb, ws)


def _mid_body(sums_ref, g_ref, b_ref, e_ref, w_ref, o_ref, st_ref):
    i = pl.program_id(0)
    s, t = _bn_st(sums_ref[...], g_ref[...], b_ref[...], float(N * K))
    h = jnp.maximum(e_ref[...].astype(jnp.float32) * s + t, 0.0)
    o = jnp.dot(h, w_ref[...])
    o_ref[...] = o.astype(jnp.bfloat16)
    _accum(st_ref, jnp.stack([jnp.sum(o, 0), jnp.sum(o * o, 0)]), i)


def _mid_layer(e_flat, sums, gamma, beta, w, block_rows=16384):
    rows, chp = e_flat.shape
    ch = w.shape[1]
    return pl.pallas_call(
        _mid_body,
        grid=(rows // block_rows,),
        in_specs=_small_specs([(chp, True), (chp, False), (chp, False)]) + [
            pl.BlockSpec((block_rows, chp), lambda i: (i, 0)),
            pl.BlockSpec((chp, ch), lambda i: (0, 0)),
        ],
        out_specs=[
            pl.BlockSpec((block_rows, ch), lambda i: (i, 0)),
            pl.BlockSpec((2, ch), lambda i: (0, 0)),
        ],
        out_shape=[
            jax.ShapeDtypeStruct((rows, ch), jnp.bfloat16),
            jax.ShapeDtypeStruct((2, ch), jnp.float32),
        ],
    )(sums, gamma, beta, e_flat, w)


def _mid_max_body(sums_ref, g_ref, b_ref, e_ref, w_ref, m_ref, st_ref):
    i = pl.program_id(0)
    s, t = _bn_st(sums_ref[...], g_ref[...], b_ref[...], float(N * K))
    h = jnp.maximum(e_ref[...].astype(jnp.float32) * s + t, 0.0)
    o = jnp.dot(h, w_ref[...])
    om = jnp.max(o.reshape(-1, N, o.shape[1]), axis=0).astype(jnp.bfloat16)

    @pl.when(i == 0)
    def _():
        m_ref[...] = om

    @pl.when(i > 0)
    def _():
        m_ref[...] = jnp.maximum(m_ref[...], om)

    _accum(st_ref, jnp.stack([jnp.sum(o, 0), jnp.sum(o * o, 0)]), i)


def _mid_max_layer(e_flat, sums, gamma, beta, w, block_rows=16384):
    rows, chp = e_flat.shape
    ch = w.shape[1]
    return pl.pallas_call(
        _mid_max_body,
        grid=(rows // block_rows,),
        in_specs=_small_specs([(chp, True), (chp, False), (chp, False)]) + [
            pl.BlockSpec((block_rows, chp), lambda i: (i, 0)),
            pl.BlockSpec((chp, ch), lambda i: (0, 0)),
        ],
        out_specs=[
            pl.BlockSpec((N, ch), lambda i: (0, 0)),
            pl.BlockSpec((2, ch), lambda i: (0, 0)),
        ],
        out_shape=[
            jax.ShapeDtypeStruct((N, ch), jnp.bfloat16),
            jax.ShapeDtypeStruct((2, ch), jnp.float32),
        ],
    )(sums, gamma, beta, e_flat, w)


def _da_body(es3_ref, g3_ref, b3_ref, ns_ref_in, gs_ref, bs_ref,
             m3_ref, skip_ref, wd_ref, wb_ref, ws_ref,
             e_ref, skip_out_ref, es_ref, ns_ref):
    g = pl.program_id(0)
    s3, t3 = _bn_st(es3_ref[...], g3_ref[...], b3_ref[...], float(N * K))
    ss, ts = _bn_st(ns_ref_in[...], gs_ref[...], bs_ref[...], float(N))
    aggr = jnp.maximum(m3_ref[...].astype(jnp.float32) * s3 + t3, 0.0)
    fts = jnp.maximum(aggr + skip_ref[...] * ss + ts, 0.0)
    skip = jnp.dot(fts, ws_ref[...])
    skip_out_ref[...] = skip
    _accum(ns_ref, jnp.stack([jnp.sum(skip, 0), jnp.sum(skip * skip, 0)]), g)
    _knn_edges_and_stats(fts, fts, wd_ref[...], wb_ref[...], e_ref, es_ref, g)


def _da_layer(es3, g3, b3, nss, gs, bs, m3, skip_prev, wd, wb, ws):
    chp = m3.shape[1]
    ch1, ch3 = wd.shape[1], ws.shape[1]
    return pl.pallas_call(
        _da_body,
        grid=(N_GRAPHS // GB,),
        in_specs=_small_specs([(chp, True), (chp, False), (chp, False),
                               (chp, True), (chp, False), (chp, False)]) + [
            pl.BlockSpec((GB * P, chp), lambda g: (g, 0)),
            pl.BlockSpec((GB * P, chp), lambda g: (g, 0)),
            pl.BlockSpec((chp, ch1), lambda g: (0, 0)),
            pl.BlockSpec((chp, ch1), lambda g: (0, 0)),
            pl.BlockSpec((chp, ch3), lambda g: (0, 0)),
        ],
        out_specs=[
            pl.BlockSpec((K, GB * P, ch1), lambda g: (0, g, 0)),
            pl.BlockSpec((GB * P, ch3), lambda g: (g, 0)),
            pl.BlockSpec((2, ch1), lambda g: (0, 0)),
            pl.BlockSpec((2, ch3), lambda g: (0, 0)),
        ],
        out_shape=[
            jax.ShapeDtypeStruct((K, N, ch1), jnp.bfloat16),
            jax.ShapeDtypeStruct((N, ch3), jnp.float32),
            jax.ShapeDtypeStruct((2, ch1), jnp.float32),
            jax.ShapeDtypeStruct((2, ch3), jnp.float32),
        ],
    )(es3, g3, b3, nss, gs, bs, m3, skip_prev, wd, wb, ws)


def _pool_body(es3_ref, g3_ref, b3_ref, ns_ref_in, gs_ref, bs_ref,
               m3_ref, skip_ref, o_ref):
    s3, t3 = _bn_st(es3_ref[...], g3_ref[...], b3_ref[...], float(N * K))
    ss, ts = _bn_st(ns_ref_in[...], gs_ref[...], bs_ref[...], float(N))
    aggr = jnp.maximum(m3_ref[...].astype(jnp.float32) * s3 + t3, 0.0)
    fts = jnp.maximum(aggr + skip_ref[...] * ss + ts, 0.0)
    o_ref[...] = jnp.mean(fts.reshape(GB, P, -1), axis=1, keepdims=True)


def _pool_layer(es3, g3, b3, nss, gs, bs, m3, skip_prev):
    chp = m3.shape[1]
    return pl.pallas_call(
        _pool_body,
        grid=(N_GRAPHS // GB,),
        in_specs=_small_specs([(chp, True), (chp, False), (chp, False),
                               (chp, True), (chp, False), (chp, False)]) + [
            pl.BlockSpec((GB * P, chp), lambda g: (g, 0)),
            pl.BlockSpec((GB * P, chp), lambda g: (g, 0)),
        ],
        out_specs=[pl.BlockSpec((GB, 1, chp), lambda g: (g, 0, 0))],
        out_shape=[jax.ShapeDtypeStruct((N_GRAPHS, 1, chp), jnp.float32)],
    )(es3, g3, b3, nss, gs, bs, m3, skip_prev)[0]


def _head_body(h_ref, wfc_ref, bfc_ref, wout_ref, bout_ref, o_ref):
    h = jnp.maximum(jnp.dot(h_ref[...], wfc_ref[...]) + bfc_ref[...], 0.0)
    logits = jnp.dot(h, wout_ref[...]) + bout_ref[...]
    m = jnp.max(logits, axis=1, keepdims=True)
    e = jnp.exp(logits - m)
    o_ref[...] = e / jnp.sum(e, axis=1, keepdims=True)


def _head(pooled, wfc, bfc, wout_pad, bout_pad):
    return pl.pallas_call(
        _head_body,
        out_shape=jax.ShapeDtypeStruct((N_GRAPHS, 128), jnp.float32),
    )(pooled, wfc, bfc, wout_pad, bout_pad)


def kernel(x, pos, batch, params):
    f = x.shape[1]
    convs = params['convs']

    prep = []
    prev = f
    for p in convs:
        w1 = p['W1']
        w1a, w1b = w1[:, :prev], w1[:, prev:]
        prep.append({
            'wd': (w1a - w1b).T, 'wb': w1b.T,
            'w2': p['W2'].T, 'w3': p['W3'].T, 'ws': p['Ws'].T,
            'g1': p['g1'].reshape(1, -1), 'b1': p['b1'].reshape(1, -1),
            'g2': p['g2'].reshape(1, -1), 'b2': p['b2'].reshape(1, -1),
            'g3': p['g3'].reshape(1, -1), 'b3': p['b3'].reshape(1, -1),
            'gs': p['gs'].reshape(1, -1), 'bs': p['bs'].reshape(1, -1),
        })
        prev = p['W3'].shape[0]

    sin = _input_stats(x)
    g0 = params['input_bn']['g'].reshape(1, -1)
    b0 = params['input_bn']['b'].reshape(1, -1)
    e1, skip, es, ns = _layer1_entry(x, pos, sin, g0, b0, prep[0]['wd'],
                                     prep[0]['wb'], prep[0]['ws'])
    for li in range(3):
        pr = prep[li]
        e2, es2 = _mid_layer(e1.reshape(N * K, -1), es, pr['g1'], pr['b1'],
                             pr['w2'])
        m3, es3 = _mid_max_layer(e2, es2, pr['g2'], pr['b2'], pr['w3'])
        if li < 2:
            e1, skip, es, ns = _da_layer(es3, pr['g3'], pr['b3'], ns,
                                         pr['gs'], pr['bs'], m3, skip,
                                         prep[li + 1]['wd'],
                                         prep[li + 1]['wb'],
                                         prep[li + 1]['ws'])
        else:
            pooled = _pool_layer(es3, pr['g3'], pr['b3'], ns,
                                 pr['gs'], pr['bs'], m3, skip)

    pooled = pooled.reshape(N_GRAPHS, -1)
    fp = params['fc'][0]
    wout = params['out']['W']
    wout_pad = jnp.zeros((wout.shape[1], 128), jnp.float32).at[:, :wout.shape[0]].set(wout.T)
    bout_pad = jnp.full((1, 128), -1e30, jnp.float32).at[0, :wout.shape[0]].set(params['out']['b'])
    probs = _head(pooled, fp['W'].T, fp['b'].reshape(1, -1), wout_pad, bout_pad)
    return probs[:, :wout.shape[0]]
